# Initial kernel scaffold; baseline (speedup 1.0000x reference)
import jax
import jax.numpy as jnp
from jax.experimental import pallas as pl


def kernel(byte_ids, byte_table, ngram_table):
    raise NotImplementedError("write your pallas kernel here")


# speedup vs baseline: 4.9961x; 4.9961x over previous
import functools

import jax
import jax.numpy as jnp
from jax import lax
from jax.experimental import pallas as pl
from jax.experimental.pallas import tpu as pltpu
import jax.experimental.pallas.tpu_sc as plsc

_NGRAM_RANGE = (3, 4, 5, 6)
_MAX_NGRAM = 6
_NGRAM_VOCAB = 65536
_DIM = 32

_NC = 2
_NS = 16
_NW = _NC * _NS
_LANES = 16

_PAD = 8
_CHUNK = 128


def _sc_body(seq_len, chunks_per_worker, bytes_hbm, btab_hbm, ntab_hbm,
             out_hbm, bytes_v, idx_v, rows_v, base_v, sem):
    tokens_per_worker = chunks_per_worker * _CHUNK
    wid = (lax.axis_index("s") * _NC + lax.axis_index("c")).astype(jnp.int32)
    workers_per_seq = seq_len // tokens_per_worker
    q = wid // workers_per_seq
    pb = (wid % workers_per_seq) * tokens_per_worker

    src_off = q * (seq_len + _PAD) + pb
    pltpu.sync_copy(bytes_hbm.at[pl.ds(src_off, tokens_per_worker + _PAD)],
                    bytes_v)

    at_seq_start = pb == 0

    for c in range(chunks_per_worker):
        for g in range(_CHUNK // _LANES):
            off = _PAD + c * _CHUNK + g * _LANES
            b0 = bytes_v[pl.ds(off, _LANES)]
            b1 = bytes_v[pl.ds(off - 1, _LANES)]
            b2 = bytes_v[pl.ds(off - 2, _LANES)]
            b3 = bytes_v[pl.ds(off - 3, _LANES)]
            b4 = bytes_v[pl.ds(off - 4, _LANES)]
            b5 = bytes_v[pl.ds(off - 5, _LANES)]
            h3 = (b0 * 513 + b1 * 257 + b2) & 0xFFFF
            h4 = (b3 + h3 * 257) & 0xFFFF
            h5 = (b4 + h4 * 257) & 0xFFFF
            h6 = (b5 + h5 * 257) & 0xFFFF
            gs = pl.ds(g * _LANES, _LANES)
            i32 = jnp.int32
            idx_v[i32(0), gs] = h3
            idx_v[i32(1), gs] = h4 + _NGRAM_VOCAB
            idx_v[i32(2), gs] = h5 + 2 * _NGRAM_VOCAB
            idx_v[i32(3), gs] = h6 + 3 * _NGRAM_VOCAB
            idx_v[i32(4), gs] = b0

        cpys = [pltpu.async_copy(
            btab_hbm.at[idx_v.at[jnp.int32(4)]], base_v, sem)]
        for r in range(4):
            cpys.append(
                pltpu.async_copy(ntab_hbm.at[idx_v.at[jnp.int32(r)]],
                                 rows_v.at[jnp.int32(r)], sem))
        for cp in cpys:
            cp.wait()

        if c == 0:
            @pl.when(at_seq_start)
            def _():
                zeros = jnp.zeros((_LANES,), jnp.float32)
                for p in range(_MAX_NGRAM - 1):
                    for r in range(4):
                        if p + 1 < _NGRAM_RANGE[r]:
                            ri, pi = jnp.int32(r), jnp.int32(p)
                            rows_v[ri, pi, pl.ds(0, _LANES)] = zeros
                            rows_v[ri, pi, pl.ds(_LANES, _LANES)] = zeros

        @pl.loop(jnp.int32(0), jnp.int32(_CHUNK))
        def _(t):
            r0, r1, r2, r3 = (jnp.int32(r) for r in range(4))
            for h in range(_DIM // _LANES):
                sl = pl.ds(h * _LANES, _LANES)
                s01 = rows_v[r0, t, sl] + rows_v[r1, t, sl]
                s23 = rows_v[r2, t, sl] + rows_v[r3, t, sl]
                base_v[t, sl] = base_v[t, sl] + (s01 + s23) * 0.25

        out0 = wid * tokens_per_worker + c * _CHUNK
        pltpu.sync_copy(base_v, out_hbm.at[pl.ds(out0, _CHUNK), :])


def kernel(byte_ids, byte_table, ngram_table):
    B, S = byte_ids.shape
    dim = byte_table.shape[-1]
    n_tokens = B * S
    tokens_per_worker = n_tokens // _NW
    chunks_per_worker = tokens_per_worker // _CHUNK

    b32 = byte_ids.astype(jnp.int32)
    bytes_ext = jnp.pad(b32, ((0, 0), (_PAD, 0))).reshape(-1)
    btab = byte_table.astype(jnp.float32)
    ntab = ngram_table.astype(jnp.float32)

    mesh = plsc.VectorSubcoreMesh(
        core_axis_name="c", subcore_axis_name="s",
        num_cores=_NC, num_subcores=_NS)

    body = functools.partial(_sc_body, S, chunks_per_worker)
    out = pl.kernel(
        body,
        out_type=jax.ShapeDtypeStruct((n_tokens, dim), jnp.float32),
        mesh=mesh,
        scratch_types=[
            pltpu.VMEM((tokens_per_worker + _PAD,), jnp.int32),
            pltpu.VMEM((5, _CHUNK), jnp.int32),
            pltpu.VMEM((4, _CHUNK, dim), jnp.float32),
            pltpu.VMEM((_CHUNK, dim), jnp.float32),
            pltpu.SemaphoreType.DMA,
        ],
        compiler_params=pltpu.CompilerParams(use_tc_tiling_on_sc=False),
    )(bytes_ext, btab, ntab)
    return out.reshape(B, S, dim)


# speedup vs baseline: 5.1985x; 1.0405x over previous
import functools

import jax
import jax.numpy as jnp
from jax import lax
from jax.experimental import pallas as pl
from jax.experimental.pallas import tpu as pltpu
import jax.experimental.pallas.tpu_sc as plsc

_NGRAM_RANGE = (3, 4, 5, 6)
_MAX_NGRAM = 6
_NGRAM_VOCAB = 65536
_DIM = 32

_NC = 2
_NS = 16
_NW = _NC * _NS
_LANES = 16

_PAD = 8
_CHUNK = 128


def _sc_body(seq_len, chunks_per_worker, bytes_hbm, btab_hbm, ntab_hbm,
             out_hbm, bytes_v, idx_v, rows_v, bbuf, obuf,
             sem_g0, sem_g1, sem_o0, sem_o1):
    i32 = jnp.int32
    tokens_per_worker = chunks_per_worker * _CHUNK
    wid = (lax.axis_index("s") * _NC + lax.axis_index("c")).astype(jnp.int32)
    workers_per_seq = seq_len // tokens_per_worker
    q = wid // workers_per_seq
    pb = (wid % workers_per_seq) * tokens_per_worker

    src_off = q * (seq_len + _PAD) + pb
    pltpu.sync_copy(bytes_hbm.at[pl.ds(src_off, tokens_per_worker + _PAD)],
                    bytes_v)

    at_seq_start = pb == 0
    sem_g = (sem_g0, sem_g1)
    sem_o = (sem_o0, sem_o1)

    def do_hash(c):
        b = i32(c % 2)
        for g in range(_CHUNK // _LANES):
            off = _PAD + c * _CHUNK + g * _LANES
            b0 = bytes_v[pl.ds(off, _LANES)]
            b1 = bytes_v[pl.ds(off - 1, _LANES)]
            b2 = bytes_v[pl.ds(off - 2, _LANES)]
            b3 = bytes_v[pl.ds(off - 3, _LANES)]
            b4 = bytes_v[pl.ds(off - 4, _LANES)]
            b5 = bytes_v[pl.ds(off - 5, _LANES)]
            h3 = (b0 * 513 + b1 * 257 + b2) & 0xFFFF
            h4 = (b3 + h3 * 257) & 0xFFFF
            h5 = (b4 + h4 * 257) & 0xFFFF
            h6 = (b5 + h5 * 257) & 0xFFFF
            gs = pl.ds(g * _LANES, _LANES)
            idx_v[b, i32(0), gs] = h3
            idx_v[b, i32(1), gs] = h4 + _NGRAM_VOCAB
            idx_v[b, i32(2), gs] = h5 + 2 * _NGRAM_VOCAB
            idx_v[b, i32(3), gs] = h6 + 3 * _NGRAM_VOCAB
            idx_v[b, i32(4), gs] = b0

    def fire_gathers(c):
        b = c % 2
        bi = i32(b)
        cpys = [pltpu.async_copy(
            btab_hbm.at[idx_v.at[bi, i32(4)]], bbuf.at[bi], sem_g[b])]
        for r in range(4):
            cpys.append(pltpu.async_copy(
                ntab_hbm.at[idx_v.at[bi, i32(r)]],
                rows_v.at[bi, i32(r)], sem_g[b]))
        return cpys

    out_cpys = {}
    do_hash(0)
    gathers = fire_gathers(0)

    for c in range(chunks_per_worker):
        b = c % 2
        bi = i32(b)

        if c + 1 < chunks_per_worker:
            do_hash(c + 1)
            next_gathers = fire_gathers(c + 1)

        for cp in gathers:
            cp.wait()
        if c + 1 < chunks_per_worker:
            gathers = next_gathers

        if c == 0:
            @pl.when(at_seq_start)
            def _():
                zeros = jnp.zeros((_LANES,), jnp.float32)
                for p in range(_MAX_NGRAM - 1):
                    for r in range(4):
                        if p + 1 < _NGRAM_RANGE[r]:
                            ri, pi = i32(r), i32(p)
                            rows_v[bi, ri, pi, pl.ds(0, _LANES)] = zeros
                            rows_v[bi, ri, pi, pl.ds(_LANES, _LANES)] = zeros

        if c >= 2:
            out_cpys.pop(c - 2).wait()

        @pl.loop(i32(0), i32(_CHUNK // 4))
        def _(tq):
            r0, r1, r2, r3 = (i32(r) for r in range(4))
            t0 = tq * i32(4)
            for k in range(4):
                t = t0 + i32(k)
                for h in range(_DIM // _LANES):
                    sl = pl.ds(h * _LANES, _LANES)
                    s01 = rows_v[bi, r0, t, sl] + rows_v[bi, r1, t, sl]
                    s23 = rows_v[bi, r2, t, sl] + rows_v[bi, r3, t, sl]
                    obuf[bi, t, sl] = bbuf[bi, t, sl] + (s01 + s23) * 0.25

        out0 = wid * tokens_per_worker + c * _CHUNK
        out_cpys[c] = pltpu.async_copy(
            obuf.at[bi], out_hbm.at[pl.ds(out0, _CHUNK), :], sem_o[b])

    for cp in out_cpys.values():
        cp.wait()


def kernel(byte_ids, byte_table, ngram_table):
    B, S = byte_ids.shape
    dim = byte_table.shape[-1]
    n_tokens = B * S
    tokens_per_worker = n_tokens // _NW
    chunks_per_worker = tokens_per_worker // _CHUNK

    b32 = byte_ids.astype(jnp.int32)
    bytes_ext = jnp.pad(b32, ((0, 0), (_PAD, 0))).reshape(-1)
    btab = byte_table.astype(jnp.float32)
    ntab = ngram_table.astype(jnp.float32)

    mesh = plsc.VectorSubcoreMesh(
        core_axis_name="c", subcore_axis_name="s",
        num_cores=_NC, num_subcores=_NS)

    body = functools.partial(_sc_body, S, chunks_per_worker)
    out = pl.kernel(
        body,
        out_type=jax.ShapeDtypeStruct((n_tokens, dim), jnp.float32),
        mesh=mesh,
        scratch_types=[
            pltpu.VMEM((tokens_per_worker + _PAD,), jnp.int32),
            pltpu.VMEM((2, 5, _CHUNK), jnp.int32),
            pltpu.VMEM((2, 4, _CHUNK, dim), jnp.float32),
            pltpu.VMEM((2, _CHUNK, dim), jnp.float32),
            pltpu.VMEM((2, _CHUNK, dim), jnp.float32),
            pltpu.SemaphoreType.DMA,
            pltpu.SemaphoreType.DMA,
            pltpu.SemaphoreType.DMA,
            pltpu.SemaphoreType.DMA,
        ],
        compiler_params=pltpu.CompilerParams(use_tc_tiling_on_sc=False),
    )(bytes_ext, btab, ntab)
    return out.reshape(B, S, dim)
